# baseline (speedup 1.0000x reference)
import functools

import jax
import jax.numpy as jnp
from jax import lax
from jax.experimental import pallas as pl
from jax.experimental.pallas import tpu as pltpu
from jax.experimental.pallas import tpu_sc as plsc

F32 = jnp.float32
_NC = 2
_NS = 16
_CH = 128
_IG = 16
_G = 64


def _sc_mesh():
    return plsc.VectorSubcoreMesh(core_axis_name="c", subcore_axis_name="s")


@functools.cache
def _make_deg_kernel(n_pad: int, p: int):
    rps = n_pad // _NS

    @functools.partial(
        pl.kernel,
        mesh=_sc_mesh(),
        out_type=jax.ShapeDtypeStruct((_NC * n_pad, 128), F32),
        scratch_types=[
            pltpu.VMEM((p, _CH), jnp.int32),
            pltpu.VMEM((_CH, 128), F32),
            pltpu.VMEM_SHARED((n_pad, 128), F32),
        ],
    )
    def deg_kernel(dst_hbm, ones_hbm, zeros_hbm, out_hbm, idxd, ones_v, acc):
        c = lax.axis_index("c")
        s = lax.axis_index("s")
        wid = c * _NS + s
        r0 = s * rps
        pltpu.sync_copy(ones_hbm, ones_v)
        pltpu.sync_copy(zeros_hbm.at[pl.ds(r0, rps)], acc.at[pl.ds(r0, rps)])
        pltpu.sync_copy(dst_hbm.at[pl.ds(wid * p, p)], idxd)
        plsc.subcore_barrier()

        @pl.loop(0, p)
        def _(j):
            pltpu.sync_copy(ones_v, acc.at[idxd.at[j]], add=True)

        plsc.subcore_barrier()
        pltpu.sync_copy(acc.at[pl.ds(r0, rps)],
                        out_hbm.at[pl.ds(c * n_pad + r0, rps)])

    return deg_kernel


@functools.cache
def _make_agg_kernel(n_pad: int, p0: int, p1: int, d: int):
    rps = n_pad // _NS

    @functools.partial(
        pl.kernel,
        mesh=_sc_mesh(),
        out_type=jax.ShapeDtypeStruct((_NC * n_pad, d), F32),
        scratch_types=[
            pltpu.VMEM((_IG, _CH), jnp.int32),
            pltpu.VMEM((_IG, _CH), jnp.int32),
            pltpu.VMEM((_CH, d), F32),
            pltpu.VMEM((_CH, d), F32),
            pltpu.VMEM_SHARED((n_pad, d), F32),
            pltpu.SemaphoreType.DMA,
            pltpu.SemaphoreType.DMA,
        ],
    )
    def agg_kernel(y_hbm, src_hbm, dst_hbm, zeros_hbm, out_hbm,
                   idxs, idxd, rows0, rows1, acc, sem0, sem1):
        c = lax.axis_index("c")
        s = lax.axis_index("s")
        r0 = s * rps
        pltpu.sync_copy(zeros_hbm.at[pl.ds(r0, rps)], acc.at[pl.ds(r0, rps)])
        plsc.subcore_barrier()

        my_base = jnp.where(c == 0, s * p0, _NS * p0 + s * p1)
        n_groups = jnp.where(c == 0, p0 // _IG, p1 // _IG)

        @pl.loop(0, n_groups)
        def _(g):
            base = my_base + g * _IG
            pltpu.sync_copy(src_hbm.at[pl.ds(base, _IG)], idxs)
            pltpu.sync_copy(dst_hbm.at[pl.ds(base, _IG)], idxd)
            pltpu.async_copy(y_hbm.at[idxs.at[0]], rows0, sem0)

            @pl.loop(0, _IG // 2)
            def _(t):
                j = 2 * t
                pltpu.async_copy(y_hbm.at[idxs.at[j + 1]], rows1, sem1)
                pltpu.make_async_copy(y_hbm.at[idxs.at[j]], rows0, sem0).wait()
                pltpu.sync_copy(rows0, acc.at[idxd.at[j]], add=True)

                @pl.when(t + 1 < _IG // 2)
                def _():
                    pltpu.async_copy(y_hbm.at[idxs.at[j + 2]], rows0, sem0)

                pltpu.make_async_copy(y_hbm.at[idxs.at[j + 1]], rows1, sem1).wait()
                pltpu.sync_copy(rows1, acc.at[idxd.at[j + 1]], add=True)

        plsc.subcore_barrier()
        pltpu.sync_copy(acc.at[pl.ds(r0, rps)],
                        out_hbm.at[pl.ds(c * n_pad + r0, rps)])

    return agg_kernel


def _matmul_body(x_ref, w_ref, o_ref):
    o_ref[...] = jnp.dot(x_ref[...], w_ref[...],
                         precision=lax.Precision.HIGHEST,
                         preferred_element_type=F32)


def _scale_body(n_pad, xw_ref, degp_ref, y_ref, dis_ref):
    degp = degp_ref[...]
    deg = 1.0 + degp[:n_pad, :1] + degp[n_pad:, :1]
    dis = lax.rsqrt(deg)
    y_ref[...] = xw_ref[...] * dis
    dis_ref[...] = jnp.broadcast_to(dis, dis_ref.shape)


def _layer2_body(n_pad, y1_ref, s_ref, dis_ref, b1_ref, w2_ref, y2_ref):
    dis = dis_ref[...][:, :1]
    z = (y1_ref[...] + s_ref[:n_pad, :] + s_ref[n_pad:, :]) * dis + b1_ref[...]
    h1 = jnp.maximum(z, 0.0)
    y2_ref[...] = jnp.dot(h1, w2_ref[...],
                          precision=lax.Precision.HIGHEST,
                          preferred_element_type=F32) * dis


def _final_body(n_pad, y2_ref, s_ref, dis_ref, b2_ref, batch_ref, wp_ref,
                bp_ref, o_ref):
    dis = dis_ref[...][:, :1]
    h2 = (y2_ref[...] + s_ref[:n_pad, :] + s_ref[n_pad:, :]) * dis + b2_ref[...]
    gids = lax.broadcasted_iota(jnp.int32, (1, _G), 1)
    onehot = (batch_ref[...] == gids).astype(F32)
    dn = (((0,), (0,)), ((), ()))
    sums = lax.dot_general(onehot, h2, dn,
                           precision=lax.Precision.HIGHEST,
                           preferred_element_type=F32)
    counts = lax.dot_general(onehot, jnp.ones((n_pad, 1), F32), dn,
                             precision=lax.Precision.HIGHEST,
                             preferred_element_type=F32)
    pooled = sums / jnp.maximum(counts, 1.0)
    o_ref[...] = jnp.dot(pooled, wp_ref[...],
                         precision=lax.Precision.HIGHEST,
                         preferred_element_type=F32) + bp_ref[...]


def kernel(x, edge_index, batch, W1, b1, W2, b2, Wp, bp):
    n, d_in = x.shape
    d_hid = W1.shape[1]
    d_out = Wp.shape[1]
    e = edge_index.shape[1]

    n_pad = ((n // 128) + 1) * 128
    block = _NC * _NS * _CH * 8
    e_pad = ((e + block - 1) // block) * block
    n_chunks = e_pad // _CH
    p = n_chunks // (_NC * _NS)
    pt = n_chunks // _NS
    p0 = pt
    p1 = 0

    src = jnp.concatenate(
        [edge_index[0], jnp.zeros((e_pad - e,), jnp.int32)]).reshape(n_chunks, _CH)
    dst = jnp.concatenate(
        [edge_index[1], jnp.full((e_pad - e,), n, jnp.int32)]).reshape(n_chunks, _CH)
    xp = jnp.pad(x, ((0, n_pad - n), (0, 0)))
    batch_p = jnp.concatenate(
        [batch, jnp.full((n_pad - n,), _G, jnp.int32)]).reshape(n_pad, 1)

    ones128 = jnp.ones((_CH, 128), F32)
    zerosd = jnp.zeros((n_pad, d_hid), F32)

    deg_k = _make_deg_kernel(n_pad, p)
    agg_k = _make_agg_kernel(n_pad, p0, p1, d_hid)

    degp = deg_k(dst, ones128, zerosd)

    xw1 = pl.pallas_call(
        _matmul_body,
        out_shape=jax.ShapeDtypeStruct((n_pad, d_hid), F32),
    )(xp, W1)

    y1, dis = pl.pallas_call(
        functools.partial(_scale_body, n_pad),
        out_shape=[jax.ShapeDtypeStruct((n_pad, d_hid), F32),
                   jax.ShapeDtypeStruct((n_pad, 16), F32)],
    )(xw1, degp)

    s1 = agg_k(y1, src, dst, zerosd)

    y2 = pl.pallas_call(
        functools.partial(_layer2_body, n_pad),
        out_shape=jax.ShapeDtypeStruct((n_pad, d_hid), F32),
    )(y1, s1, dis, b1.reshape(1, -1), W2)

    s2 = agg_k(y2, src, dst, zerosd)

    out = pl.pallas_call(
        functools.partial(_final_body, n_pad),
        out_shape=jax.ShapeDtypeStruct((_G, d_out), F32),
    )(y2, s2, dis, b2.reshape(1, -1), batch_p, Wp, bp.reshape(1, -1))

    return out


# speedup vs baseline: 2.9771x; 2.9771x over previous
import functools

import jax
import jax.numpy as jnp
from jax import lax
from jax.experimental import pallas as pl
from jax.experimental.pallas import tpu as pltpu
from jax.experimental.pallas import tpu_sc as plsc

F32 = jnp.float32
_NC = 2
_NS = 16
_CH = 128
_IG = 16
_G = 64


def _sc_mesh():
    return plsc.VectorSubcoreMesh(core_axis_name="c", subcore_axis_name="s")


@functools.cache
def _make_deg_kernel(n_pad: int, p: int):
    rps = n_pad // _NS

    @functools.partial(
        pl.kernel,
        mesh=_sc_mesh(),
        out_type=jax.ShapeDtypeStruct((_NC * n_pad, 128), F32),
        scratch_types=[
            pltpu.VMEM((p, _CH), jnp.int32),
            pltpu.VMEM((_CH, 128), F32),
            pltpu.VMEM_SHARED((n_pad, 128), F32),
        ],
    )
    def deg_kernel(dst_hbm, ones_hbm, zeros_hbm, out_hbm, idxd, ones_v, acc):
        c = lax.axis_index("c")
        s = lax.axis_index("s")
        wid = c * _NS + s
        r0 = s * rps
        pltpu.sync_copy(ones_hbm, ones_v)
        pltpu.sync_copy(zeros_hbm.at[pl.ds(r0, rps)], acc.at[pl.ds(r0, rps)])
        pltpu.sync_copy(dst_hbm.at[pl.ds(wid * p, p)], idxd)
        plsc.subcore_barrier()

        @pl.loop(0, p)
        def _(j):
            pltpu.sync_copy(ones_v, acc.at[idxd.at[j]], add=True)

        plsc.subcore_barrier()
        pltpu.sync_copy(acc.at[pl.ds(r0, rps)],
                        out_hbm.at[pl.ds(c * n_pad + r0, rps)])

    return deg_kernel


@functools.cache
def _make_agg_kernel(n_pad: int, p0: int, p1: int, d: int):
    rps = n_pad // _NS

    @functools.partial(
        pl.kernel,
        mesh=_sc_mesh(),
        out_type=jax.ShapeDtypeStruct((_NC * n_pad, d), F32),
        scratch_types=[
            pltpu.VMEM((_IG, _CH), jnp.int32),
            pltpu.VMEM((_IG, _CH), jnp.int32),
            pltpu.VMEM((_CH, d), F32),
            pltpu.VMEM((_CH, d), F32),
            pltpu.VMEM_SHARED((n_pad, d), F32),
            pltpu.SemaphoreType.DMA,
            pltpu.SemaphoreType.DMA,
        ],
    )
    def agg_kernel(y_hbm, src_hbm, dst_hbm, zeros_hbm, out_hbm,
                   idxs, idxd, rows0, rows1, acc, sem0, sem1):
        c = lax.axis_index("c")
        s = lax.axis_index("s")
        r0 = s * rps
        pltpu.sync_copy(zeros_hbm.at[pl.ds(r0, rps)], acc.at[pl.ds(r0, rps)])
        plsc.subcore_barrier()

        my_base = jnp.where(c == 0, s * p0, _NS * p0 + s * p1)
        n_groups = jnp.where(c == 0, p0 // _IG, p1 // _IG)

        @pl.loop(0, n_groups)
        def _(g):
            base = my_base + g * _IG
            pltpu.sync_copy(src_hbm.at[pl.ds(base, _IG)], idxs)
            pltpu.sync_copy(dst_hbm.at[pl.ds(base, _IG)], idxd)
            pltpu.async_copy(y_hbm.at[idxs.at[0]], rows0, sem0)

            @pl.loop(0, _IG // 2)
            def _(t):
                j = 2 * t
                pltpu.async_copy(y_hbm.at[idxs.at[j + 1]], rows1, sem1)
                pltpu.make_async_copy(y_hbm.at[idxs.at[j]], rows0, sem0).wait()
                pltpu.sync_copy(rows0, acc.at[idxd.at[j]], add=True)

                @pl.when(t + 1 < _IG // 2)
                def _():
                    pltpu.async_copy(y_hbm.at[idxs.at[j + 2]], rows0, sem0)

                pltpu.make_async_copy(y_hbm.at[idxs.at[j + 1]], rows1, sem1).wait()
                pltpu.sync_copy(rows1, acc.at[idxd.at[j + 1]], add=True)

        plsc.subcore_barrier()
        pltpu.sync_copy(acc.at[pl.ds(r0, rps)],
                        out_hbm.at[pl.ds(c * n_pad + r0, rps)])

    return agg_kernel


def _matmul_body(x_ref, w_ref, o_ref):
    o_ref[...] = jnp.dot(x_ref[...], w_ref[...],
                         precision=lax.Precision.HIGHEST,
                         preferred_element_type=F32)


def _scale_body(n_pad, xw_ref, degp_ref, y_ref, dis_ref):
    degp = degp_ref[...]
    deg = 1.0 + degp[:n_pad, :1] + degp[n_pad:, :1]
    dis = lax.rsqrt(deg)
    y_ref[...] = xw_ref[...] * dis
    dis_ref[...] = jnp.broadcast_to(dis, dis_ref.shape)


def _layer2_body(n_pad, y1_ref, s_ref, dis_ref, b1_ref, w2_ref, y2_ref):
    dis = dis_ref[...][:, :1]
    z = (y1_ref[...] + s_ref[:n_pad, :] + s_ref[n_pad:, :]) * dis + b1_ref[...]
    h1 = jnp.maximum(z, 0.0)
    y2_ref[...] = jnp.dot(h1, w2_ref[...],
                          precision=lax.Precision.HIGHEST,
                          preferred_element_type=F32) * dis


def _final_body(n_pad, y2_ref, s_ref, dis_ref, b2_ref, batch_ref, wp_ref,
                bp_ref, o_ref):
    dis = dis_ref[...][:, :1]
    h2 = (y2_ref[...] + s_ref[:n_pad, :] + s_ref[n_pad:, :]) * dis + b2_ref[...]
    gids = lax.broadcasted_iota(jnp.int32, (1, _G), 1)
    onehot = (batch_ref[...] == gids).astype(F32)
    dn = (((0,), (0,)), ((), ()))
    sums = lax.dot_general(onehot, h2, dn,
                           precision=lax.Precision.HIGHEST,
                           preferred_element_type=F32)
    counts = lax.dot_general(onehot, jnp.ones((n_pad, 1), F32), dn,
                             precision=lax.Precision.HIGHEST,
                             preferred_element_type=F32)
    pooled = sums / jnp.maximum(counts, 1.0)
    o_ref[...] = jnp.dot(pooled, wp_ref[...],
                         precision=lax.Precision.HIGHEST,
                         preferred_element_type=F32) + bp_ref[...]


def kernel(x, edge_index, batch, W1, b1, W2, b2, Wp, bp):
    n, d_in = x.shape
    d_hid = W1.shape[1]
    d_out = Wp.shape[1]
    e = edge_index.shape[1]

    n_pad = ((n // 128) + 1) * 128
    block = _NC * _NS * _CH * 8
    e_pad = ((e + block - 1) // block) * block
    n_chunks = e_pad // _CH
    p = n_chunks // (_NC * _NS)
    pt = n_chunks // _NS
    p0 = pt // 2
    p1 = pt - p0

    pad_idx = n + (jnp.arange(e_pad - e, dtype=jnp.int32) % (n_pad - n))
    src = jnp.concatenate([edge_index[0], pad_idx]).reshape(n_chunks, _CH)
    dst = jnp.concatenate([edge_index[1], pad_idx]).reshape(n_chunks, _CH)
    xp = jnp.pad(x, ((0, n_pad - n), (0, 0)))
    batch_p = jnp.concatenate(
        [batch, jnp.full((n_pad - n,), _G, jnp.int32)]).reshape(n_pad, 1)

    ones128 = jnp.ones((_CH, 128), F32)
    zerosd = jnp.zeros((n_pad, d_hid), F32)

    deg_k = _make_deg_kernel(n_pad, p)
    agg_k = _make_agg_kernel(n_pad, p0, p1, d_hid)

    degp = deg_k(dst, ones128, zerosd)

    xw1 = pl.pallas_call(
        _matmul_body,
        out_shape=jax.ShapeDtypeStruct((n_pad, d_hid), F32),
    )(xp, W1)

    y1, dis = pl.pallas_call(
        functools.partial(_scale_body, n_pad),
        out_shape=[jax.ShapeDtypeStruct((n_pad, d_hid), F32),
                   jax.ShapeDtypeStruct((n_pad, 16), F32)],
    )(xw1, degp)

    s1 = agg_k(y1, src, dst, zerosd)

    y2 = pl.pallas_call(
        functools.partial(_layer2_body, n_pad),
        out_shape=jax.ShapeDtypeStruct((n_pad, d_hid), F32),
    )(y1, s1, dis, b1.reshape(1, -1), W2)

    s2 = agg_k(y2, src, dst, zerosd)

    out = pl.pallas_call(
        functools.partial(_final_body, n_pad),
        out_shape=jax.ShapeDtypeStruct((_G, d_out), F32),
    )(y2, s2, dis, b2.reshape(1, -1), batch_p, Wp, bp.reshape(1, -1))

    return out


# speedup vs baseline: 3.0993x; 1.0410x over previous
import functools

import jax
import jax.numpy as jnp
from jax import lax
from jax.experimental import pallas as pl
from jax.experimental.pallas import tpu as pltpu
from jax.experimental.pallas import tpu_sc as plsc

F32 = jnp.float32
_NC = 2
_NS = 16
_CH = 128
_IG = 40
_G = 64


def _sc_mesh():
    return plsc.VectorSubcoreMesh(core_axis_name="c", subcore_axis_name="s")


@functools.cache
def _make_deg_kernel(n_pad: int, p: int):
    rps = n_pad // _NS

    @functools.partial(
        pl.kernel,
        mesh=_sc_mesh(),
        out_type=jax.ShapeDtypeStruct((_NC * n_pad, 128), F32),
        scratch_types=[
            pltpu.VMEM((p, _CH), jnp.int32),
            pltpu.VMEM((_CH, 128), F32),
            pltpu.VMEM_SHARED((n_pad, 128), F32),
        ],
    )
    def deg_kernel(dst_hbm, ones_hbm, zeros_hbm, out_hbm, idxd, ones_v, acc):
        c = lax.axis_index("c")
        s = lax.axis_index("s")
        wid = c * _NS + s
        r0 = s * rps
        pltpu.sync_copy(ones_hbm, ones_v)
        pltpu.sync_copy(zeros_hbm.at[pl.ds(r0, rps)], acc.at[pl.ds(r0, rps)])
        pltpu.sync_copy(dst_hbm.at[pl.ds(wid * p, p)], idxd)
        plsc.subcore_barrier()

        @pl.loop(0, p)
        def _(j):
            pltpu.sync_copy(ones_v, acc.at[idxd.at[j]], add=True)

        plsc.subcore_barrier()
        pltpu.sync_copy(acc.at[pl.ds(r0, rps)],
                        out_hbm.at[pl.ds(c * n_pad + r0, rps)])

    return deg_kernel


@functools.cache
def _make_agg_kernel(n_pad: int, p0: int, p1: int, d: int):
    rps = n_pad // _NS

    @functools.partial(
        pl.kernel,
        mesh=_sc_mesh(),
        out_type=jax.ShapeDtypeStruct((_NC * n_pad, d), F32),
        scratch_types=[
            pltpu.VMEM((_IG, _CH), jnp.int32),
            pltpu.VMEM((_IG, _CH), jnp.int32),
            pltpu.VMEM((_CH, d), F32),
            pltpu.VMEM((_CH, d), F32),
            pltpu.VMEM_SHARED((n_pad, d), F32),
            pltpu.SemaphoreType.DMA,
            pltpu.SemaphoreType.DMA,
        ],
    )
    def agg_kernel(y_hbm, src_hbm, dst_hbm, zeros_hbm, out_hbm,
                   idxs, idxd, rows0, rows1, acc, sem0, sem1):
        c = lax.axis_index("c")
        s = lax.axis_index("s")
        r0 = s * rps
        pltpu.sync_copy(zeros_hbm.at[pl.ds(r0, rps)], acc.at[pl.ds(r0, rps)])
        plsc.subcore_barrier()

        my_base = jnp.where(c == 0, s * p0, _NS * p0 + s * p1)
        n_groups = jnp.where(c == 0, p0 // _IG, p1 // _IG)

        @pl.loop(0, n_groups)
        def _(g):
            base = my_base + g * _IG
            pltpu.sync_copy(src_hbm.at[pl.ds(base, _IG)], idxs)
            pltpu.sync_copy(dst_hbm.at[pl.ds(base, _IG)], idxd)
            pltpu.async_copy(y_hbm.at[idxs.at[0]], rows0, sem0)

            @pl.loop(0, _IG // 2)
            def _(t):
                j = 2 * t
                pltpu.async_copy(y_hbm.at[idxs.at[j + 1]], rows1, sem1)
                pltpu.make_async_copy(y_hbm.at[idxs.at[j]], rows0, sem0).wait()
                pltpu.sync_copy(rows0, acc.at[idxd.at[j]], add=True)

                @pl.when(t + 1 < _IG // 2)
                def _():
                    pltpu.async_copy(y_hbm.at[idxs.at[j + 2]], rows0, sem0)

                pltpu.make_async_copy(y_hbm.at[idxs.at[j + 1]], rows1, sem1).wait()
                pltpu.sync_copy(rows1, acc.at[idxd.at[j + 1]], add=True)

        plsc.subcore_barrier()
        pltpu.sync_copy(acc.at[pl.ds(r0, rps)],
                        out_hbm.at[pl.ds(c * n_pad + r0, rps)])

    return agg_kernel


def _matmul_body(x_ref, w_ref, o_ref):
    o_ref[...] = jnp.dot(x_ref[...], w_ref[...],
                         precision=lax.Precision.HIGHEST,
                         preferred_element_type=F32)


def _scale_body(n_pad, xw_ref, degp_ref, y_ref, dis_ref):
    degp = degp_ref[...]
    deg = 1.0 + degp[:n_pad, :1] + degp[n_pad:, :1]
    dis = lax.rsqrt(deg)
    y_ref[...] = xw_ref[...] * dis
    dis_ref[...] = jnp.broadcast_to(dis, dis_ref.shape)


def _layer2_body(n_pad, y1_ref, s_ref, dis_ref, b1_ref, w2_ref, y2_ref):
    dis = dis_ref[...][:, :1]
    z = (y1_ref[...] + s_ref[:n_pad, :] + s_ref[n_pad:, :]) * dis + b1_ref[...]
    h1 = jnp.maximum(z, 0.0)
    y2_ref[...] = jnp.dot(h1, w2_ref[...],
                          precision=lax.Precision.HIGHEST,
                          preferred_element_type=F32) * dis


def _final_body(n_pad, y2_ref, s_ref, dis_ref, b2_ref, batch_ref, wp_ref,
                bp_ref, o_ref):
    dis = dis_ref[...][:, :1]
    h2 = (y2_ref[...] + s_ref[:n_pad, :] + s_ref[n_pad:, :]) * dis + b2_ref[...]
    gids = lax.broadcasted_iota(jnp.int32, (1, _G), 1)
    onehot = (batch_ref[...] == gids).astype(F32)
    dn = (((0,), (0,)), ((), ()))
    sums = lax.dot_general(onehot, h2, dn,
                           precision=lax.Precision.HIGHEST,
                           preferred_element_type=F32)
    counts = lax.dot_general(onehot, jnp.ones((n_pad, 1), F32), dn,
                             precision=lax.Precision.HIGHEST,
                             preferred_element_type=F32)
    pooled = sums / jnp.maximum(counts, 1.0)
    o_ref[...] = jnp.dot(pooled, wp_ref[...],
                         precision=lax.Precision.HIGHEST,
                         preferred_element_type=F32) + bp_ref[...]


def kernel(x, edge_index, batch, W1, b1, W2, b2, Wp, bp):
    n, d_in = x.shape
    d_hid = W1.shape[1]
    d_out = Wp.shape[1]
    e = edge_index.shape[1]

    n_pad = ((n // 128) + 1) * 128
    block = _NC * _NS * _CH * 8
    e_pad = ((e + block - 1) // block) * block
    n_chunks = e_pad // _CH
    p = n_chunks // (_NC * _NS)
    pt = n_chunks // _NS
    p0 = pt // 2
    p1 = pt - p0

    pad_idx = n + (jnp.arange(e_pad - e, dtype=jnp.int32) % (n_pad - n))
    src = jnp.concatenate([edge_index[0], pad_idx]).reshape(n_chunks, _CH)
    dst = jnp.concatenate([edge_index[1], pad_idx]).reshape(n_chunks, _CH)
    xp = jnp.pad(x, ((0, n_pad - n), (0, 0)))
    batch_p = jnp.concatenate(
        [batch, jnp.full((n_pad - n,), _G, jnp.int32)]).reshape(n_pad, 1)

    ones128 = jnp.ones((_CH, 128), F32)
    zerosd = jnp.zeros((n_pad, d_hid), F32)

    deg_k = _make_deg_kernel(n_pad, p)
    agg_k = _make_agg_kernel(n_pad, p0, p1, d_hid)

    degp = deg_k(dst, ones128, zerosd)

    xw1 = pl.pallas_call(
        _matmul_body,
        out_shape=jax.ShapeDtypeStruct((n_pad, d_hid), F32),
    )(xp, W1)

    y1, dis = pl.pallas_call(
        functools.partial(_scale_body, n_pad),
        out_shape=[jax.ShapeDtypeStruct((n_pad, d_hid), F32),
                   jax.ShapeDtypeStruct((n_pad, 16), F32)],
    )(xw1, degp)

    s1 = agg_k(y1, src, dst, zerosd)

    y2 = pl.pallas_call(
        functools.partial(_layer2_body, n_pad),
        out_shape=jax.ShapeDtypeStruct((n_pad, d_hid), F32),
    )(y1, s1, dis, b1.reshape(1, -1), W2)

    s2 = agg_k(y2, src, dst, zerosd)

    out = pl.pallas_call(
        functools.partial(_final_body, n_pad),
        out_shape=jax.ShapeDtypeStruct((_G, d_out), F32),
    )(y2, s2, dis, b2.reshape(1, -1), batch_p, Wp, bp.reshape(1, -1))

    return out


# speedup vs baseline: 3.7526x; 1.2108x over previous
import dataclasses
import functools

import jax
import jax.numpy as jnp
from jax import lax
from jax.experimental import pallas as pl
from jax.experimental.pallas import tpu as pltpu
from jax.experimental.pallas import tpu_sc as plsc

F32 = jnp.float32
_NC = 2
_NS = 16
_CH = 128
_IG = 40
_G = 64


def _sc_mesh():
    return plsc.VectorSubcoreMesh(core_axis_name="c", subcore_axis_name="s")


def _sc_vector_params():
    cp = pltpu.CompilerParams()
    if "needs_layout_passes" in pltpu.CompilerParams.__dataclass_fields__:
        cp = dataclasses.replace(cp, needs_layout_passes=False)
    return cp


@functools.cache
def _make_deg_kernel(n_pad: int, p: int):
    assert n_pad <= 128 * 128

    @functools.partial(
        pl.kernel,
        mesh=_sc_mesh(),
        out_type=jax.ShapeDtypeStruct((_NC * 128, 128), F32),
        compiler_params=_sc_vector_params(),
        scratch_types=[
            pltpu.VMEM((p, _CH), jnp.int32),
            pltpu.VMEM((128, 128), F32),
            pltpu.VMEM((1, 128), jnp.int32),
            pltpu.VMEM_SHARED((128, 128), F32),
        ],
    )
    def deg_kernel(dst_hbm, out_hbm, idxd, tbl, idxrow, acc):
        c = lax.axis_index("c")
        s = lax.axis_index("s")
        wid = c * _NS + s

        @pl.loop(0, 128)
        def _(r):
            for l in range(8):
                tbl[r, pl.ds(l * 16, 16)] = jnp.zeros((16,), F32)

        for k in range(8):
            idxrow[0, pl.ds(k * 16, 16)] = lax.iota(jnp.int32, 16) + 16 * k

        pltpu.sync_copy(tbl.at[pl.ds(0, 8)], acc.at[pl.ds(s * 8, 8)])
        pltpu.sync_copy(dst_hbm.at[pl.ds(wid * p, p)], idxd)
        ones_v = jnp.ones((16,), F32)

        @pl.loop(0, p)
        def _(j):
            for k in range(8):
                idx = idxd[j, pl.ds(k * 16, 16)]
                hi = lax.shift_right_logical(idx, 7)
                lo = jnp.bitwise_and(idx, 127)
                plsc.addupdate_scatter(tbl, [hi, lo], ones_v)

        plsc.subcore_barrier()
        pltpu.sync_copy(tbl, acc.at[idxrow.at[0]], add=True)
        plsc.subcore_barrier()
        pltpu.sync_copy(acc.at[pl.ds(s * 8, 8)],
                        out_hbm.at[pl.ds(c * 128 + s * 8, 8)])

    return deg_kernel


@functools.cache
def _make_agg_kernel(n_pad: int, p0: int, p1: int, d: int):
    rps = n_pad // _NS

    @functools.partial(
        pl.kernel,
        mesh=_sc_mesh(),
        out_type=jax.ShapeDtypeStruct((_NC * n_pad, d), F32),
        scratch_types=[
            pltpu.VMEM((_IG, _CH), jnp.int32),
            pltpu.VMEM((_IG, _CH), jnp.int32),
            pltpu.VMEM((_CH, d), F32),
            pltpu.VMEM((_CH, d), F32),
            pltpu.VMEM_SHARED((n_pad, d), F32),
            pltpu.SemaphoreType.DMA,
            pltpu.SemaphoreType.DMA,
        ],
    )
    def agg_kernel(y_hbm, src_hbm, dst_hbm, zeros_hbm, out_hbm,
                   idxs, idxd, rows0, rows1, acc, sem0, sem1):
        c = lax.axis_index("c")
        s = lax.axis_index("s")
        r0 = s * rps
        pltpu.sync_copy(zeros_hbm.at[pl.ds(r0, rps)], acc.at[pl.ds(r0, rps)])
        plsc.subcore_barrier()

        my_base = jnp.where(c == 0, s * p0, _NS * p0 + s * p1)
        n_groups = jnp.where(c == 0, p0 // _IG, p1 // _IG)

        @pl.loop(0, n_groups)
        def _(g):
            base = my_base + g * _IG
            pltpu.sync_copy(src_hbm.at[pl.ds(base, _IG)], idxs)
            pltpu.sync_copy(dst_hbm.at[pl.ds(base, _IG)], idxd)
            pltpu.async_copy(y_hbm.at[idxs.at[0]], rows0, sem0)

            @pl.loop(0, _IG // 2)
            def _(t):
                j = 2 * t
                pltpu.async_copy(y_hbm.at[idxs.at[j + 1]], rows1, sem1)
                pltpu.make_async_copy(y_hbm.at[idxs.at[j]], rows0, sem0).wait()
                pltpu.sync_copy(rows0, acc.at[idxd.at[j]], add=True)

                @pl.when(t + 1 < _IG // 2)
                def _():
                    pltpu.async_copy(y_hbm.at[idxs.at[j + 2]], rows0, sem0)

                pltpu.make_async_copy(y_hbm.at[idxs.at[j + 1]], rows1, sem1).wait()
                pltpu.sync_copy(rows1, acc.at[idxd.at[j + 1]], add=True)

        plsc.subcore_barrier()
        pltpu.sync_copy(acc.at[pl.ds(r0, rps)],
                        out_hbm.at[pl.ds(c * n_pad + r0, rps)])

    return agg_kernel


def _matmul_body(x_ref, w_ref, o_ref):
    o_ref[...] = jnp.dot(x_ref[...], w_ref[...],
                         precision=lax.Precision.HIGHEST,
                         preferred_element_type=F32)


def _scale_body(n_pad, xw_ref, degp_ref, y_ref, dis_ref):
    eye = (lax.broadcasted_iota(jnp.int32, (128, 128), 0)
           == lax.broadcasted_iota(jnp.int32, (128, 128), 1)).astype(F32)

    def body(b, _):
        drow = lax.rsqrt(1.0 + degp_ref[pl.ds(b, 1), :]
                         + degp_ref[pl.ds(128 + b, 1), :])
        dcol = jnp.sum(eye * drow, axis=1, keepdims=True)
        blk = xw_ref[pl.ds(b * 128, 128), :]
        y_ref[pl.ds(b * 128, 128), :] = blk * dcol
        dis_ref[pl.ds(b * 128, 128), :] = jnp.broadcast_to(dcol, (128, 16))
        return 0

    lax.fori_loop(0, n_pad // 128, body, 0)


def _layer2_body(n_pad, y1_ref, s_ref, dis_ref, b1_ref, w2_ref, y2_ref):
    dis = dis_ref[...][:, :1]
    z = (y1_ref[...] + s_ref[:n_pad, :] + s_ref[n_pad:, :]) * dis + b1_ref[...]
    h1 = jnp.maximum(z, 0.0)
    y2_ref[...] = jnp.dot(h1, w2_ref[...],
                          precision=lax.Precision.HIGHEST,
                          preferred_element_type=F32) * dis


def _final_body(n_pad, y2_ref, s_ref, dis_ref, b2_ref, batch_ref, wp_ref,
                bp_ref, o_ref):
    dis = dis_ref[...][:, :1]
    h2 = (y2_ref[...] + s_ref[:n_pad, :] + s_ref[n_pad:, :]) * dis + b2_ref[...]
    gids = lax.broadcasted_iota(jnp.int32, (1, _G), 1)
    onehot = (batch_ref[...] == gids).astype(F32)
    dn = (((0,), (0,)), ((), ()))
    sums = lax.dot_general(onehot, h2, dn,
                           precision=lax.Precision.HIGHEST,
                           preferred_element_type=F32)
    counts = lax.dot_general(onehot, jnp.ones((n_pad, 1), F32), dn,
                             precision=lax.Precision.HIGHEST,
                             preferred_element_type=F32)
    pooled = sums / jnp.maximum(counts, 1.0)
    o_ref[...] = jnp.dot(pooled, wp_ref[...],
                         precision=lax.Precision.HIGHEST,
                         preferred_element_type=F32) + bp_ref[...]


def kernel(x, edge_index, batch, W1, b1, W2, b2, Wp, bp):
    n, d_in = x.shape
    d_hid = W1.shape[1]
    d_out = Wp.shape[1]
    e = edge_index.shape[1]

    n_pad = ((n // 128) + 1) * 128
    block = _NC * _NS * _CH * 8
    e_pad = ((e + block - 1) // block) * block
    n_chunks = e_pad // _CH
    p = n_chunks // (_NC * _NS)
    pt = n_chunks // _NS
    p0 = pt // 2
    p1 = pt - p0

    pad_idx = n + (jnp.arange(e_pad - e, dtype=jnp.int32) % (n_pad - n))
    src = jnp.concatenate([edge_index[0], pad_idx]).reshape(n_chunks, _CH)
    dst = jnp.concatenate([edge_index[1], pad_idx]).reshape(n_chunks, _CH)
    xp = jnp.pad(x, ((0, n_pad - n), (0, 0)))
    batch_p = jnp.concatenate(
        [batch, jnp.full((n_pad - n,), _G, jnp.int32)]).reshape(n_pad, 1)

    zerosd = jnp.zeros((n_pad, d_hid), F32)

    deg_k = _make_deg_kernel(n_pad, p)
    agg_k = _make_agg_kernel(n_pad, p0, p1, d_hid)

    degp = deg_k(dst)

    xw1 = pl.pallas_call(
        _matmul_body,
        out_shape=jax.ShapeDtypeStruct((n_pad, d_hid), F32),
    )(xp, W1)

    y1, dis = pl.pallas_call(
        functools.partial(_scale_body, n_pad),
        out_shape=[jax.ShapeDtypeStruct((n_pad, d_hid), F32),
                   jax.ShapeDtypeStruct((n_pad, 16), F32)],
    )(xw1, degp)

    s1 = agg_k(y1, src, dst, zerosd)

    y2 = pl.pallas_call(
        functools.partial(_layer2_body, n_pad),
        out_shape=jax.ShapeDtypeStruct((n_pad, d_hid), F32),
    )(y1, s1, dis, b1.reshape(1, -1), W2)

    s2 = agg_k(y2, src, dst, zerosd)

    out = pl.pallas_call(
        functools.partial(_final_body, n_pad),
        out_shape=jax.ShapeDtypeStruct((_G, d_out), F32),
    )(y2, s2, dis, b2.reshape(1, -1), batch_p, Wp, bp.reshape(1, -1))

    return out


# speedup vs baseline: 3.8901x; 1.0366x over previous
import dataclasses
import functools

import jax
import jax.numpy as jnp
from jax import lax
from jax.experimental import pallas as pl
from jax.experimental.pallas import tpu as pltpu
from jax.experimental.pallas import tpu_sc as plsc

F32 = jnp.float32
_NC = 2
_NS = 16
_CH = 128
_IG = 40
_G = 64


def _sc_mesh():
    return plsc.VectorSubcoreMesh(core_axis_name="c", subcore_axis_name="s")


def _sc_vector_params():
    cp = pltpu.CompilerParams()
    if "needs_layout_passes" in pltpu.CompilerParams.__dataclass_fields__:
        cp = dataclasses.replace(cp, needs_layout_passes=False)
    return cp


@functools.cache
def _make_deg_kernel(n_pad: int, p: int):
    assert n_pad <= 128 * 128

    @functools.partial(
        pl.kernel,
        mesh=_sc_mesh(),
        out_type=jax.ShapeDtypeStruct((_NC * 128, 128), F32),
        compiler_params=_sc_vector_params(),
        scratch_types=[
            pltpu.VMEM((p, _CH), jnp.int32),
            pltpu.VMEM((128, 128), F32),
            pltpu.VMEM((1, 128), jnp.int32),
            pltpu.VMEM_SHARED((128, 128), F32),
        ],
    )
    def deg_kernel(dst_hbm, out_hbm, idxd, tbl, idxrow, acc):
        c = lax.axis_index("c")
        s = lax.axis_index("s")
        wid = c * _NS + s

        @pl.loop(0, 128)
        def _(r):
            for l in range(8):
                tbl[r, pl.ds(l * 16, 16)] = jnp.zeros((16,), F32)

        for k in range(8):
            idxrow[0, pl.ds(k * 16, 16)] = lax.iota(jnp.int32, 16) + 16 * k

        pltpu.sync_copy(tbl.at[pl.ds(0, 8)], acc.at[pl.ds(s * 8, 8)])
        pltpu.sync_copy(dst_hbm.at[pl.ds(wid * p, p)], idxd)
        ones_v = jnp.ones((16,), F32)

        @pl.loop(0, p)
        def _(j):
            for k in range(8):
                idx = idxd[j, pl.ds(k * 16, 16)]
                hi = lax.shift_right_logical(idx, 7)
                lo = jnp.bitwise_and(idx, 127)
                plsc.addupdate_scatter(tbl, [hi, lo], ones_v)

        plsc.subcore_barrier()
        pltpu.sync_copy(tbl, acc.at[idxrow.at[0]], add=True)
        plsc.subcore_barrier()
        pltpu.sync_copy(acc.at[pl.ds(s * 8, 8)],
                        out_hbm.at[pl.ds(c * 128 + s * 8, 8)])

    return deg_kernel


@functools.cache
def _make_agg_kernel(n_pad: int, p0: int, p1: int, d: int):
    rps = n_pad // _NS

    @functools.partial(
        pl.kernel,
        mesh=_sc_mesh(),
        out_type=jax.ShapeDtypeStruct((_NC * n_pad, d), F32),
        scratch_types=[
            pltpu.VMEM((_IG, _CH), jnp.int32),
            pltpu.VMEM((_IG, _CH), jnp.int32),
            pltpu.VMEM((_CH, d), F32),
            pltpu.VMEM((_CH, d), F32),
            pltpu.VMEM_SHARED((n_pad, d), F32),
            pltpu.SemaphoreType.DMA,
            pltpu.SemaphoreType.DMA,
        ],
    )
    def agg_kernel(y_hbm, src_hbm, dst_hbm, zeros_hbm, out_hbm,
                   idxs, idxd, rows0, rows1, acc, sem0, sem1):
        c = lax.axis_index("c")
        s = lax.axis_index("s")
        r0 = s * rps
        pltpu.sync_copy(zeros_hbm.at[pl.ds(r0, rps)], acc.at[pl.ds(r0, rps)])
        plsc.subcore_barrier()

        my_base = jnp.where(c == 0, s * p0, _NS * p0 + s * p1)
        n_groups = jnp.where(c == 0, p0 // _IG, p1 // _IG)

        @pl.loop(0, n_groups)
        def _(g):
            base = my_base + g * _IG
            pltpu.sync_copy(src_hbm.at[pl.ds(base, _IG)], idxs)
            pltpu.sync_copy(dst_hbm.at[pl.ds(base, _IG)], idxd)
            pltpu.async_copy(y_hbm.at[idxs.at[0]], rows0, sem0)

            @pl.loop(0, _IG // 2)
            def _(t):
                j = 2 * t
                pltpu.async_copy(y_hbm.at[idxs.at[j + 1]], rows1, sem1)
                pltpu.make_async_copy(y_hbm.at[idxs.at[j]], rows0, sem0).wait()
                pltpu.sync_copy(rows0, acc.at[idxd.at[j]], add=True)

                @pl.when(t + 1 < _IG // 2)
                def _():
                    pltpu.async_copy(y_hbm.at[idxs.at[j + 2]], rows0, sem0)

                pltpu.make_async_copy(y_hbm.at[idxs.at[j + 1]], rows1, sem1).wait()
                pltpu.sync_copy(rows1, acc.at[idxd.at[j + 1]], add=True)

        plsc.subcore_barrier()
        pltpu.sync_copy(acc.at[pl.ds(r0, rps)],
                        out_hbm.at[pl.ds(c * n_pad + r0, rps)])

    return agg_kernel


def _matmul_body(x_ref, w_ref, o_ref):
    o_ref[...] = jnp.dot(x_ref[...], w_ref[...],
                         precision=lax.Precision.DEFAULT,
                         preferred_element_type=F32)


def _scale_body(n_pad, xw_ref, degp_ref, y_ref, dis_ref):
    degp = degp_ref[...]
    dis2d = lax.rsqrt(1.0 + degp[:128, :] + degp[128:, :])
    eye = (lax.broadcasted_iota(jnp.int32, (128, 128), 0)
           == lax.broadcasted_iota(jnp.int32, (128, 128), 1)).astype(F32)
    dis_t = lax.dot_general(eye, dis2d, (((1,), (1,)), ((), ())),
                            precision=lax.Precision.DEFAULT,
                            preferred_element_type=F32)
    for b in range(n_pad // 128):
        dcol = dis_t[:, b:b + 1]
        y_ref[pl.ds(b * 128, 128), :] = xw_ref[pl.ds(b * 128, 128), :] * dcol
        dis_ref[pl.ds(b * 128, 128), :] = jnp.broadcast_to(dcol, (128, 16))


def _layer2_body(n_pad, y1_ref, s_ref, dis_ref, b1_ref, w2_ref, y2_ref):
    dis = dis_ref[...][:, :1]
    z = (y1_ref[...] + s_ref[:n_pad, :] + s_ref[n_pad:, :]) * dis + b1_ref[...]
    h1 = jnp.maximum(z, 0.0)
    y2_ref[...] = jnp.dot(h1, w2_ref[...],
                          precision=lax.Precision.DEFAULT,
                          preferred_element_type=F32) * dis


def _final_body(n_pad, y2_ref, s_ref, dis_ref, b2_ref, batch_ref, wp_ref,
                bp_ref, o_ref):
    dis = dis_ref[...][:, :1]
    h2 = (y2_ref[...] + s_ref[:n_pad, :] + s_ref[n_pad:, :]) * dis + b2_ref[...]
    gids = lax.broadcasted_iota(jnp.int32, (1, _G), 1)
    onehot = (batch_ref[...] == gids).astype(F32)
    dn = (((0,), (0,)), ((), ()))
    sums = lax.dot_general(onehot, h2, dn,
                           precision=lax.Precision.DEFAULT,
                           preferred_element_type=F32)
    counts = lax.dot_general(onehot, jnp.ones((n_pad, 1), F32), dn,
                             precision=lax.Precision.DEFAULT,
                             preferred_element_type=F32)
    pooled = sums / jnp.maximum(counts, 1.0)
    o_ref[...] = jnp.dot(pooled, wp_ref[...],
                         precision=lax.Precision.DEFAULT,
                         preferred_element_type=F32) + bp_ref[...]


def kernel(x, edge_index, batch, W1, b1, W2, b2, Wp, bp):
    n, d_in = x.shape
    d_hid = W1.shape[1]
    d_out = Wp.shape[1]
    e = edge_index.shape[1]

    n_pad = ((n // 128) + 1) * 128
    block = _NC * _NS * _CH * 8
    e_pad = ((e + block - 1) // block) * block
    n_chunks = e_pad // _CH
    p = n_chunks // (_NC * _NS)
    pt = n_chunks // _NS
    p0 = pt // 2
    p1 = pt - p0

    assert e % _CH == 0
    e3 = edge_index.reshape(2, e // _CH, _CH)
    ar = jnp.arange((n_chunks - e // _CH) * _CH, dtype=jnp.int32)
    pad_block = (n + ar % (n_pad - n)).reshape(-1, _CH)
    src = jnp.concatenate([e3[0], pad_block], axis=0)
    dst = jnp.concatenate([e3[1], pad_block], axis=0)
    xp = jnp.pad(x, ((0, n_pad - n), (0, 0)))
    batch_p = jnp.concatenate(
        [batch, jnp.full((n_pad - n,), _G, jnp.int32)]).reshape(n_pad, 1)

    zerosd = jnp.zeros((n_pad, d_hid), F32)

    deg_k = _make_deg_kernel(n_pad, p)
    agg_k = _make_agg_kernel(n_pad, p0, p1, d_hid)

    degp = deg_k(dst)

    xw1 = pl.pallas_call(
        _matmul_body,
        out_shape=jax.ShapeDtypeStruct((n_pad, d_hid), F32),
    )(xp, W1)

    y1, dis = pl.pallas_call(
        functools.partial(_scale_body, n_pad),
        out_shape=[jax.ShapeDtypeStruct((n_pad, d_hid), F32),
                   jax.ShapeDtypeStruct((n_pad, 16), F32)],
    )(xw1, degp)

    s1 = agg_k(y1, src, dst, zerosd)

    y2 = pl.pallas_call(
        functools.partial(_layer2_body, n_pad),
        out_shape=jax.ShapeDtypeStruct((n_pad, d_hid), F32),
    )(y1, s1, dis, b1.reshape(1, -1), W2)

    s2 = agg_k(y2, src, dst, zerosd)

    out = pl.pallas_call(
        functools.partial(_final_body, n_pad),
        out_shape=jax.ShapeDtypeStruct((_G, d_out), F32),
    )(y2, s2, dis, b2.reshape(1, -1), batch_p, Wp, bp.reshape(1, -1))

    return out


# speedup vs baseline: 4.0066x; 1.0299x over previous
import dataclasses
import functools

import jax
import jax.numpy as jnp
from jax import lax
from jax.experimental import pallas as pl
from jax.experimental.pallas import tpu as pltpu
from jax.experimental.pallas import tpu_sc as plsc

F32 = jnp.float32
_NC = 2
_NS = 16
_CH = 128
_IG = 40
_G = 64


def _sc_mesh():
    return plsc.VectorSubcoreMesh(core_axis_name="c", subcore_axis_name="s")


def _sc_vector_params():
    cp = pltpu.CompilerParams()
    if "needs_layout_passes" in pltpu.CompilerParams.__dataclass_fields__:
        cp = dataclasses.replace(cp, needs_layout_passes=False)
    return cp


@functools.cache
def _make_deg_kernel(n_pad: int, p: int):
    assert n_pad <= 128 * 128

    @functools.partial(
        pl.kernel,
        mesh=_sc_mesh(),
        out_type=jax.ShapeDtypeStruct((_NC * 128, 128), F32),
        compiler_params=_sc_vector_params(),
        scratch_types=[
            pltpu.VMEM((p, _CH), jnp.int32),
            pltpu.VMEM((128, 128), F32),
            pltpu.VMEM((1, 128), jnp.int32),
            pltpu.VMEM_SHARED((128, 128), F32),
        ],
    )
    def deg_kernel(dst_hbm, out_hbm, idxd, tbl, idxrow, acc):
        c = lax.axis_index("c")
        s = lax.axis_index("s")
        wid = c * _NS + s

        @pl.loop(0, 128)
        def _(r):
            for l in range(8):
                tbl[r, pl.ds(l * 16, 16)] = jnp.zeros((16,), F32)

        for k in range(8):
            idxrow[0, pl.ds(k * 16, 16)] = lax.iota(jnp.int32, 16) + 16 * k

        pltpu.sync_copy(tbl.at[pl.ds(0, 8)], acc.at[pl.ds(s * 8, 8)])
        pltpu.sync_copy(dst_hbm.at[pl.ds(wid * p, p)], idxd)
        ones_v = jnp.ones((16,), F32)

        @pl.loop(0, p)
        def _(j):
            for k in range(8):
                idx = idxd[j, pl.ds(k * 16, 16)]
                hi = lax.shift_right_logical(idx, 7)
                lo = jnp.bitwise_and(idx, 127)
                plsc.addupdate_scatter(tbl, [hi, lo], ones_v)

        plsc.subcore_barrier()
        pltpu.sync_copy(tbl, acc.at[idxrow.at[0]], add=True)
        plsc.subcore_barrier()
        pltpu.sync_copy(acc.at[pl.ds(s * 8, 8)],
                        out_hbm.at[pl.ds(c * 128 + s * 8, 8)])

    return deg_kernel


@functools.cache
def _make_agg_kernel(n_pad: int, p0: int, p1: int, d: int):
    rps = n_pad // _NS

    @functools.partial(
        pl.kernel,
        mesh=_sc_mesh(),
        out_type=jax.ShapeDtypeStruct((_NC * n_pad, d), F32),
        compiler_params=_sc_vector_params(),
        scratch_types=[
            pltpu.VMEM((_IG, _CH), jnp.int32),
            pltpu.VMEM((_IG, _CH), jnp.int32),
            pltpu.VMEM((_CH, d), F32),
            pltpu.VMEM((_CH, d), F32),
            pltpu.VMEM_SHARED((n_pad, d), F32),
            pltpu.SemaphoreType.DMA,
            pltpu.SemaphoreType.DMA,
        ],
    )
    def agg_kernel(y_hbm, src_hbm, dst_hbm, out_hbm,
                   idxs, idxd, rows0, rows1, acc, sem0, sem1):
        c = lax.axis_index("c")
        s = lax.axis_index("s")
        r0 = s * rps

        @pl.loop(0, _CH)
        def _(r):
            for l in range(d // 16):
                rows0[r, pl.ds(l * 16, 16)] = jnp.zeros((16,), F32)

        off = 0
        while off < rps:
            step = min(_CH, rps - off)
            pltpu.sync_copy(rows0.at[pl.ds(0, step)],
                            acc.at[pl.ds(r0 + off, step)])
            off += step
        plsc.subcore_barrier()

        my_base = jnp.where(c == 0, s * p0, _NS * p0 + s * p1)
        n_groups = jnp.where(c == 0, p0 // _IG, p1 // _IG)

        @pl.loop(0, n_groups)
        def _(g):
            base = my_base + g * _IG
            pltpu.sync_copy(src_hbm.at[pl.ds(base, _IG)], idxs)
            pltpu.sync_copy(dst_hbm.at[pl.ds(base, _IG)], idxd)
            pltpu.async_copy(y_hbm.at[idxs.at[0]], rows0, sem0)

            @pl.loop(0, _IG // 2)
            def _(t):
                j = 2 * t
                pltpu.async_copy(y_hbm.at[idxs.at[j + 1]], rows1, sem1)
                pltpu.make_async_copy(y_hbm.at[idxs.at[j]], rows0, sem0).wait()
                pltpu.sync_copy(rows0, acc.at[idxd.at[j]], add=True)

                @pl.when(t + 1 < _IG // 2)
                def _():
                    pltpu.async_copy(y_hbm.at[idxs.at[j + 2]], rows0, sem0)

                pltpu.make_async_copy(y_hbm.at[idxs.at[j + 1]], rows1, sem1).wait()
                pltpu.sync_copy(rows1, acc.at[idxd.at[j + 1]], add=True)

        plsc.subcore_barrier()
        pltpu.sync_copy(acc.at[pl.ds(r0, rps)],
                        out_hbm.at[pl.ds(c * n_pad + r0, rps)])

    return agg_kernel


def _matmul_body(x_ref, w_ref, o_ref):
    o_ref[...] = jnp.dot(x_ref[...], w_ref[...],
                         precision=lax.Precision.DEFAULT,
                         preferred_element_type=F32)


def _scale_body(n_pad, xw_ref, degp_ref, y_ref, dis_ref):
    degp = degp_ref[...]
    dis2d = lax.rsqrt(1.0 + degp[:128, :] + degp[128:, :])
    eye = (lax.broadcasted_iota(jnp.int32, (128, 128), 0)
           == lax.broadcasted_iota(jnp.int32, (128, 128), 1)).astype(F32)
    dis_t = lax.dot_general(eye, dis2d, (((1,), (1,)), ((), ())),
                            precision=lax.Precision.DEFAULT,
                            preferred_element_type=F32)
    for b in range(n_pad // 128):
        dcol = dis_t[:, b:b + 1]
        y_ref[pl.ds(b * 128, 128), :] = xw_ref[pl.ds(b * 128, 128), :] * dcol
        dis_ref[pl.ds(b * 128, 128), :] = jnp.broadcast_to(dcol, (128, 16))


def _layer2_body(n_pad, y1_ref, s_ref, dis_ref, b1_ref, w2_ref, y2_ref):
    dis = dis_ref[...][:, :1]
    z = (y1_ref[...] + s_ref[:n_pad, :] + s_ref[n_pad:, :]) * dis + b1_ref[...]
    h1 = jnp.maximum(z, 0.0)
    y2_ref[...] = jnp.dot(h1, w2_ref[...],
                          precision=lax.Precision.DEFAULT,
                          preferred_element_type=F32) * dis


def _final_body(n_pad, y2_ref, s_ref, dis_ref, b2_ref, batch_ref, wp_ref,
                bp_ref, o_ref):
    dis = dis_ref[...][:, :1]
    h2 = (y2_ref[...] + s_ref[:n_pad, :] + s_ref[n_pad:, :]) * dis + b2_ref[...]
    gids = lax.broadcasted_iota(jnp.int32, (1, _G), 1)
    onehot = (batch_ref[...] == gids).astype(F32)
    dn = (((0,), (0,)), ((), ()))
    sums = lax.dot_general(onehot, h2, dn,
                           precision=lax.Precision.DEFAULT,
                           preferred_element_type=F32)
    counts = lax.dot_general(onehot, jnp.ones((n_pad, 1), F32), dn,
                             precision=lax.Precision.DEFAULT,
                             preferred_element_type=F32)
    pooled = sums / jnp.maximum(counts, 1.0)
    o_ref[...] = jnp.dot(pooled, wp_ref[...],
                         precision=lax.Precision.DEFAULT,
                         preferred_element_type=F32) + bp_ref[...]


def kernel(x, edge_index, batch, W1, b1, W2, b2, Wp, bp):
    n, d_in = x.shape
    d_hid = W1.shape[1]
    d_out = Wp.shape[1]
    e = edge_index.shape[1]

    n_pad = ((n // 128) + 1) * 128
    block = _NC * _NS * _CH * 8
    e_pad = ((e + block - 1) // block) * block
    n_chunks = e_pad // _CH
    p = n_chunks // (_NC * _NS)
    pt = n_chunks // _NS
    p0 = pt // 2
    p1 = pt - p0

    assert e % _CH == 0
    e3 = edge_index.reshape(2, e // _CH, _CH)
    ar = jnp.arange((n_chunks - e // _CH) * _CH, dtype=jnp.int32)
    pad_block = (n + ar % (n_pad - n)).reshape(-1, _CH)
    src = jnp.concatenate([e3[0], pad_block], axis=0)
    dst = jnp.concatenate([e3[1], pad_block], axis=0)
    xp = jnp.pad(x, ((0, n_pad - n), (0, 0)))
    batch_p = jnp.concatenate(
        [batch, jnp.full((n_pad - n,), _G, jnp.int32)]).reshape(n_pad, 1)

    deg_k = _make_deg_kernel(n_pad, p)
    agg_k = _make_agg_kernel(n_pad, p0, p1, d_hid)

    degp = deg_k(dst)

    xw1 = pl.pallas_call(
        _matmul_body,
        out_shape=jax.ShapeDtypeStruct((n_pad, d_hid), F32),
    )(xp, W1)

    y1, dis = pl.pallas_call(
        functools.partial(_scale_body, n_pad),
        out_shape=[jax.ShapeDtypeStruct((n_pad, d_hid), F32),
                   jax.ShapeDtypeStruct((n_pad, 16), F32)],
    )(xw1, degp)

    s1 = agg_k(y1, src, dst)

    y2 = pl.pallas_call(
        functools.partial(_layer2_body, n_pad),
        out_shape=jax.ShapeDtypeStruct((n_pad, d_hid), F32),
    )(y1, s1, dis, b1.reshape(1, -1), W2)

    s2 = agg_k(y2, src, dst)

    out = pl.pallas_call(
        functools.partial(_final_body, n_pad),
        out_shape=jax.ShapeDtypeStruct((_G, d_out), F32),
    )(y2, s2, dis, b2.reshape(1, -1), batch_p, Wp, bp.reshape(1, -1))

    return out
